# baseline (speedup 1.0000x reference)
import jax
import jax.numpy as jnp
from jax import lax
from jax.experimental import pallas as pl
from jax.experimental.pallas import tpu as pltpu
from jax.experimental.pallas import tpu_sc as plsc

VOCAB = 1000000
EMBED_DIM = 64
BATCH = 16384

NC = 2
NS = 16
NW = NC * NS

CHUNK = 128
IDX_PER_W = 2 * BATCH // NW
IDX_ROWS_PER_W = IDX_PER_W // CHUNK
N_PASS = 2
ROWS_PER_PASS = IDX_ROWS_PER_W // N_PASS
IDX_PER_PASS = ROWS_PER_PASS * CHUNK

VB = 16384
R_GRID = (VOCAB + VB - 1) // VB


def _relayout_body(src_ref, dst_ref):
    x = src_ref[...]
    y = x.T.reshape(VB // 16, 16, EMBED_DIM)
    lo = y[:, :8, :].reshape(VB // 2, EMBED_DIM)
    hi = y[:, 8:, :].reshape(VB // 2, EMBED_DIM)
    dst_ref[...] = jnp.concatenate([lo, hi], axis=1)


@jax.jit
def _pair_relayout(tableT):
    return pl.pallas_call(
        _relayout_body,
        grid=(R_GRID,),
        in_specs=[pl.BlockSpec((EMBED_DIM, VB), lambda i: (0, i))],
        out_specs=pl.BlockSpec((VB // 2, 2 * EMBED_DIM), lambda i: (i, 0)),
        out_shape=jax.ShapeDtypeStruct((VOCAB // 2, 2 * EMBED_DIM),
                                       jnp.float32),
    )(tableT)


def _do_work(pidx_hbm, table2_hbm, pairs_hbm, idx_v, rows_v, sem, k):
    pltpu.sync_copy(pidx_hbm.at[pl.ds(k * IDX_ROWS_PER_W, IDX_ROWS_PER_W)],
                    idx_v)
    for s in range(N_PASS):
        copies = [
            pltpu.async_copy(
                table2_hbm.at[idx_v.at[s * ROWS_PER_PASS + j]],
                rows_v.at[pl.ds(j * CHUNK, CHUNK)],
                sem,
            )
            for j in range(ROWS_PER_PASS)
        ]
        for c in copies:
            c.wait()
        pltpu.sync_copy(
            rows_v,
            pairs_hbm.at[pl.ds(k * IDX_PER_W + s * IDX_PER_PASS,
                               IDX_PER_PASS)])


def _gather_body(pidx_t_hbm, pidx_o_hbm, table2_hbm, pairs_t_hbm,
                 pairs_o_hbm, idx_v, rows_v, sem):
    wid = lax.axis_index("s") * NC + lax.axis_index("c")

    @pl.when(wid < NW // 2)
    def _():
        _do_work(pidx_t_hbm, table2_hbm, pairs_t_hbm, idx_v, rows_v, sem,
                 wid)

    @pl.when(wid >= NW // 2)
    def _():
        _do_work(pidx_o_hbm, table2_hbm, pairs_o_hbm, idx_v, rows_v, sem,
                 wid - NW // 2)


@jax.jit
def _sc_gather(pidx_t, pidx_o, table2):
    mesh = plsc.VectorSubcoreMesh(core_axis_name="c", subcore_axis_name="s")
    out_sds = jax.ShapeDtypeStruct((BATCH, 2 * EMBED_DIM), jnp.float32)
    run = pl.kernel(
        _gather_body,
        out_type=(out_sds, out_sds),
        mesh=mesh,
        compiler_params=pltpu.CompilerParams(use_tc_tiling_on_sc=True),
        scratch_types=[
            pltpu.VMEM((IDX_ROWS_PER_W, CHUNK), jnp.int32),
            pltpu.VMEM((IDX_PER_PASS, 2 * EMBED_DIM), jnp.float32),
            pltpu.SemaphoreType.DMA,
        ],
    )
    return run(pidx_t, pidx_o, table2)


SB = 512


def _select_body(pairs_ref, mask_ref, outT_ref):
    p = pairs_ref[...]
    m = mask_ref[..., :EMBED_DIM]
    lo = p[:, :EMBED_DIM]
    hi = p[:, EMBED_DIM:]
    outT_ref[...] = (lo + (hi - lo) * m).T


@jax.jit
def _select_transpose(pairs, mask):
    return pl.pallas_call(
        _select_body,
        grid=(BATCH // SB,),
        in_specs=[pl.BlockSpec((SB, 2 * EMBED_DIM), lambda i: (i, 0)),
                  pl.BlockSpec((SB, 2 * EMBED_DIM), lambda i: (i, 0))],
        out_specs=pl.BlockSpec((EMBED_DIM, SB), lambda i: (0, i)),
        out_shape=jax.ShapeDtypeStruct((EMBED_DIM, BATCH), jnp.float32),
    )(pairs, mask)


def kernel(target, other, table):
    target = target.astype(jnp.int32)
    other = other.astype(jnp.int32)
    def _pid(idx):
        return (lax.shift_right_logical(idx, 4) * 8
                + jnp.bitwise_and(idx, 7)).reshape(BATCH // CHUNK, CHUNK)

    def _mask(idx):
        half = jnp.bitwise_and(lax.shift_right_logical(idx, 3), 1)
        return jnp.broadcast_to(half.astype(jnp.float32)[:, None],
                                (BATCH, 2 * EMBED_DIM))

    pidx_t = _pid(target)
    pidx_o = _pid(other)
    mask_t = _mask(target)
    mask_o = _mask(other)

    table2 = _pair_relayout(table.T)
    pairs_t, pairs_o = _sc_gather(pidx_t, pidx_o, table2)
    out_t = _select_transpose(pairs_t, mask_t).T
    out_o = _select_transpose(pairs_o, mask_o).T
    return (out_t, out_o)


# speedup vs baseline: 1.0475x; 1.0475x over previous
import jax
import jax.numpy as jnp
from jax import lax
from jax.experimental import pallas as pl
from jax.experimental.pallas import tpu as pltpu
from jax.experimental.pallas import tpu_sc as plsc

VOCAB = 1000000
EMBED_DIM = 64
BATCH = 16384

NC = 2
NS = 16
NW = NC * NS

CHUNK = 128
IDX_PER_W = 2 * BATCH // NW
IDX_ROWS_PER_W = IDX_PER_W // CHUNK
N_PASS = 2
ROWS_PER_PASS = IDX_ROWS_PER_W // N_PASS
IDX_PER_PASS = ROWS_PER_PASS * CHUNK

VB = 32768
R_GRID = (VOCAB + VB - 1) // VB


def _relayout_body(src_ref, dst_ref):
    x = src_ref[...]
    y = x.T.reshape(VB // 16, 16, EMBED_DIM)
    lo = y[:, :8, :].reshape(VB // 2, EMBED_DIM)
    hi = y[:, 8:, :].reshape(VB // 2, EMBED_DIM)
    dst_ref[...] = jnp.concatenate([lo, hi], axis=1)


@jax.jit
def _pair_relayout(tableT):
    return pl.pallas_call(
        _relayout_body,
        grid=(R_GRID,),
        in_specs=[pl.BlockSpec((EMBED_DIM, VB), lambda i: (0, i))],
        out_specs=pl.BlockSpec((VB // 2, 2 * EMBED_DIM), lambda i: (i, 0)),
        out_shape=jax.ShapeDtypeStruct((VOCAB // 2, 2 * EMBED_DIM),
                                       jnp.float32),
    )(tableT)


def _do_work(pidx_hbm, table2_hbm, pairs_hbm, idx_v, rows_v, sem, k):
    pltpu.sync_copy(pidx_hbm.at[pl.ds(k * IDX_ROWS_PER_W, IDX_ROWS_PER_W)],
                    idx_v)
    for s in range(N_PASS):
        copies = [
            pltpu.async_copy(
                table2_hbm.at[idx_v.at[s * ROWS_PER_PASS + j]],
                rows_v.at[pl.ds(j * CHUNK, CHUNK)],
                sem,
            )
            for j in range(ROWS_PER_PASS)
        ]
        for c in copies:
            c.wait()
        pltpu.sync_copy(
            rows_v,
            pairs_hbm.at[pl.ds(k * IDX_PER_W + s * IDX_PER_PASS,
                               IDX_PER_PASS)])


def _gather_body(pidx_t_hbm, pidx_o_hbm, table2_hbm, pairs_t_hbm,
                 pairs_o_hbm, idx_v, rows_v, sem):
    wid = lax.axis_index("s") * NC + lax.axis_index("c")

    @pl.when(wid < NW // 2)
    def _():
        _do_work(pidx_t_hbm, table2_hbm, pairs_t_hbm, idx_v, rows_v, sem,
                 wid)

    @pl.when(wid >= NW // 2)
    def _():
        _do_work(pidx_o_hbm, table2_hbm, pairs_o_hbm, idx_v, rows_v, sem,
                 wid - NW // 2)


@jax.jit
def _sc_gather(pidx_t, pidx_o, table2):
    mesh = plsc.VectorSubcoreMesh(core_axis_name="c", subcore_axis_name="s")
    out_sds = jax.ShapeDtypeStruct((BATCH, 2 * EMBED_DIM), jnp.float32)
    run = pl.kernel(
        _gather_body,
        out_type=(out_sds, out_sds),
        mesh=mesh,
        compiler_params=pltpu.CompilerParams(use_tc_tiling_on_sc=True),
        scratch_types=[
            pltpu.VMEM((IDX_ROWS_PER_W, CHUNK), jnp.int32),
            pltpu.VMEM((IDX_PER_PASS, 2 * EMBED_DIM), jnp.float32),
            pltpu.SemaphoreType.DMA,
        ],
    )
    return run(pidx_t, pidx_o, table2)


SB = 512


def _select_body(pairs_ref, mask_ref, outT_ref):
    p = pairs_ref[...]
    m = mask_ref[..., :EMBED_DIM]
    lo = p[:, :EMBED_DIM]
    hi = p[:, EMBED_DIM:]
    outT_ref[...] = (lo + (hi - lo) * m).T


@jax.jit
def _select_transpose(pairs, mask):
    return pl.pallas_call(
        _select_body,
        grid=(BATCH // SB,),
        in_specs=[pl.BlockSpec((SB, 2 * EMBED_DIM), lambda i: (i, 0)),
                  pl.BlockSpec((SB, 2 * EMBED_DIM), lambda i: (i, 0))],
        out_specs=pl.BlockSpec((EMBED_DIM, SB), lambda i: (0, i)),
        out_shape=jax.ShapeDtypeStruct((EMBED_DIM, BATCH), jnp.float32),
    )(pairs, mask)


def kernel(target, other, table):
    target = target.astype(jnp.int32)
    other = other.astype(jnp.int32)
    def _pid(idx):
        return (lax.shift_right_logical(idx, 4) * 8
                + jnp.bitwise_and(idx, 7)).reshape(BATCH // CHUNK, CHUNK)

    def _mask(idx):
        half = jnp.bitwise_and(lax.shift_right_logical(idx, 3), 1)
        return jnp.broadcast_to(half.astype(jnp.float32)[:, None],
                                (BATCH, 2 * EMBED_DIM))

    pidx_t = _pid(target)
    pidx_o = _pid(other)
    mask_t = _mask(target)
    mask_o = _mask(other)

    table2 = _pair_relayout(table.T)
    pairs_t, pairs_o = _sc_gather(pidx_t, pidx_o, table2)
    out_t = _select_transpose(pairs_t, mask_t).T
    out_o = _select_transpose(pairs_o, mask_o).T
    return (out_t, out_o)
